# baseline (speedup 1.0000x reference)
import functools

import jax
import jax.numpy as jnp
from jax import lax
from jax.experimental import pallas as pl
from jax.experimental.pallas import tpu as pltpu
from jax.experimental.pallas import tpu_sc as plsc

_N = 65536
_NACT = 12
_EMB = 48
_D = (_NACT + 1) * _EMB

_NC = 2
_NSUB = 16
_NW = _NC * _NSUB
_NSC = 12288
_RPW = _NSC // _NW
_CH = 64
_NCHUNK = _RPW // _CH
_GR = 16
_NG = _CH // _GR


def _sc_body(x_hbm, wb_hbm, fin_hbm, fout_hbm, wbuf, xb0, xb1, ofin, ofout, sem0, sem1):
    cid = lax.axis_index("c")
    sid = lax.axis_index("s")
    wid = sid * _NC + cid
    rowbase = wid * _RPW

    pltpu.sync_copy(wb_hbm, wbuf)
    nw = _EMB * _NACT + _NACT
    scal = []
    for k in range((nw + _GR - 1) // _GR):
        wv = wbuf[pl.ds(k * _GR, _GR)]
        for l in range(_GR):
            if k * _GR + l < nw:
                scal.append(wv[l])

    def chunk_src(c):
        return x_hbm.at[pl.ds((rowbase + c * _CH) * _D, _CH * _D)]

    lanes = jnp.arange(_GR, dtype=jnp.int32) * (_D + 1)

    def do_chunk(c, buf, sem, nbuf, nsem):
        nxt = jnp.minimum(c + 1, _NCHUNK - 1)
        pltpu.async_copy(chunk_src(nxt), nbuf, nsem)
        pltpu.make_async_copy(chunk_src(c), buf, sem).wait()

        def group(g, carry):
            idx0 = lanes + g * (_GR * _D)
            acc_out = [None] * _NACT
            acc_in = [None] * _NACT
            for e in range(_EMB):
                x0e = plsc.load_gather(buf, [idx0 + e])
                for j in range(_NACT):
                    w = scal[e * _NACT + j]
                    xje = plsc.load_gather(buf, [idx0 + (j + 1) * _EMB + e])
                    if e == 0:
                        acc_out[j] = x0e * w
                        acc_in[j] = xje * w
                    else:
                        acc_out[j] = acc_out[j] + x0e * w
                        acc_in[j] = acc_in[j] + xje * w
            fout_v = None
            fin_v = None
            for j in range(_NACT):
                bj = scal[_EMB * _NACT + j]
                eo = jnp.exp(acc_out[j] + bj)
                ei = jnp.exp(acc_in[j] + bj)
                fout_v = eo if fout_v is None else fout_v + eo
                fin_v = ei if fin_v is None else fin_v + ei
            ofin[pl.ds(g * _GR, _GR)] = fin_v
            ofout[pl.ds(g * _GR, _GR)] = fout_v
            return carry

        lax.fori_loop(0, _NG, group, 0)
        pltpu.sync_copy(ofin, fin_hbm.at[pl.ds(rowbase + c * _CH, _CH)])
        pltpu.sync_copy(ofout, fout_hbm.at[pl.ds(rowbase + c * _CH, _CH)])

    pltpu.async_copy(chunk_src(0), xb0, sem0)

    def outer(c2, carry):
        do_chunk(2 * c2, xb0, sem0, xb1, sem1)
        do_chunk(2 * c2 + 1, xb1, sem1, xb0, sem0)
        return carry

    lax.fori_loop(0, _NCHUNK // 2, outer, 0)
    pltpu.make_async_copy(chunk_src(_NCHUNK - 1), xb0, sem0).wait()


@functools.partial(jax.jit, static_argnames=())
def _sc_flow(x1d, wb):
    f = pl.kernel(
        _sc_body,
        out_type=[
            jax.ShapeDtypeStruct((_NSC,), jnp.float32),
            jax.ShapeDtypeStruct((_NSC,), jnp.float32),
        ],
        mesh=plsc.VectorSubcoreMesh(core_axis_name="c", subcore_axis_name="s"),
        compiler_params=pltpu.CompilerParams(
            needs_layout_passes=False,
            disable_bounds_checks=True,
        ),
        scratch_types=[
            pltpu.VMEM((((_EMB * _NACT + _NACT + _GR - 1) // _GR) * _GR,), jnp.float32),
            pltpu.VMEM((_CH * _D,), jnp.float32),
            pltpu.VMEM((_CH * _D,), jnp.float32),
            pltpu.VMEM((_CH,), jnp.float32),
            pltpu.VMEM((_CH,), jnp.float32),
            pltpu.SemaphoreType.DMA,
            pltpu.SemaphoreType.DMA,
        ],
    )
    return f(x1d, wb)


def _flow_body(x_ref, w_ref, b_ref, s_ref, o_ref):
    x = x_ref[...]
    y = jnp.dot(x, w_ref[...], preferred_element_type=jnp.float32)
    y = jnp.exp(y + b_ref[...])
    o_ref[...] = jnp.dot(y, s_ref[...], preferred_element_type=jnp.float32)


def _build_wbig(W, b):
    eye = jnp.eye(_NACT, dtype=W.dtype)
    top = jnp.concatenate([W, jnp.zeros((_EMB, _NACT), W.dtype)], axis=1)
    low = (W.T[:, :, None] * eye[:, None, :]).reshape(_NACT * _EMB, _NACT)
    low = jnp.concatenate([jnp.zeros((_NACT * _EMB, _NACT), W.dtype), low], axis=1)
    wbig = jnp.concatenate([top, low], axis=0)
    bbig = jnp.concatenate([b, b])[None, :]
    ones = jnp.ones((_NACT, 1), W.dtype)
    zs = jnp.zeros((_NACT, 1), W.dtype)
    sel = jnp.concatenate(
        [
            jnp.concatenate([zs, ones], axis=1),
            jnp.concatenate([ones, zs], axis=1),
        ],
        axis=0,
    )
    return wbig, bbig, sel


def _tc_flow(xt, W, b):
    nt = _N - _NSC
    wbig, bbig, sel = _build_wbig(W, b)
    bn = 2048
    return pl.pallas_call(
        _flow_body,
        grid=(nt // bn,),
        in_specs=[
            pl.BlockSpec((bn, _D), lambda i: (i, 0)),
            pl.BlockSpec((_D, 2 * _NACT), lambda i: (0, 0)),
            pl.BlockSpec((1, 2 * _NACT), lambda i: (0, 0)),
            pl.BlockSpec((2 * _NACT, 2), lambda i: (0, 0)),
        ],
        out_specs=pl.BlockSpec((bn, 2), lambda i: (i, 0)),
        out_shape=jax.ShapeDtypeStruct((nt, 2), jnp.float32),
        compiler_params=pltpu.CompilerParams(
            dimension_semantics=("arbitrary",),
        ),
    )(xt, wbig, bbig, sel)


def kernel(inputs, W, b):
    x1d = inputs[:_NSC].reshape(-1)
    nw = _EMB * _NACT + _NACT
    npad = ((nw + _GR - 1) // _GR) * _GR
    wb = jnp.concatenate(
        [W.reshape(-1), b, jnp.zeros((npad - nw,), jnp.float32)]
    )
    fin, fout = _sc_flow(x1d, wb)
    xt = inputs[_NSC:].reshape(_N - _NSC, _D)
    tc_out = _tc_flow(xt, W, b)
    return jnp.concatenate([jnp.stack([fin, fout], axis=1), tc_out], axis=0)


# speedup vs baseline: 1.8462x; 1.8462x over previous
import jax
import jax.numpy as jnp
from jax.experimental import pallas as pl
from jax.experimental.pallas import tpu as pltpu

_N = 65536
_NACT = 12
_EMB = 48
_D = (_NACT + 1) * _EMB


def _flow_body(x_ref, w_ref, b_ref, s_ref, o_ref):
    x = x_ref[...]
    y = jnp.dot(x, w_ref[...], preferred_element_type=jnp.float32)
    y = jnp.exp(y + b_ref[...])
    o_ref[...] = jnp.dot(y, s_ref[...], preferred_element_type=jnp.float32)


def _build_wbig(W, b):
    eye = jnp.eye(_NACT, dtype=W.dtype)
    top = jnp.concatenate([W, jnp.zeros((_EMB, _NACT), W.dtype)], axis=1)
    low = (W.T[:, :, None] * eye[:, None, :]).reshape(_NACT * _EMB, _NACT)
    low = jnp.concatenate([jnp.zeros((_NACT * _EMB, _NACT), W.dtype), low], axis=1)
    wbig = jnp.concatenate([top, low], axis=0)
    bbig = jnp.concatenate([b, b])[None, :]
    ones = jnp.ones((_NACT, 1), W.dtype)
    zs = jnp.zeros((_NACT, 1), W.dtype)
    sel = jnp.concatenate(
        [
            jnp.concatenate([zs, ones], axis=1),
            jnp.concatenate([ones, zs], axis=1),
        ],
        axis=0,
    )
    return wbig, bbig, sel


def kernel(inputs, W, b):
    x = inputs.reshape(_N, _D)
    wbig, bbig, sel = _build_wbig(W, b)
    bn = 4096
    grid = (_N // bn,)
    out = pl.pallas_call(
        _flow_body,
        grid=grid,
        in_specs=[
            pl.BlockSpec((bn, _D), lambda i: (i, 0)),
            pl.BlockSpec((_D, 2 * _NACT), lambda i: (0, 0)),
            pl.BlockSpec((1, 2 * _NACT), lambda i: (0, 0)),
            pl.BlockSpec((2 * _NACT, 2), lambda i: (0, 0)),
        ],
        out_specs=pl.BlockSpec((bn, 2), lambda i: (i, 0)),
        out_shape=jax.ShapeDtypeStruct((_N, 2), jnp.float32),
        compiler_params=pltpu.CompilerParams(
            dimension_semantics=("arbitrary",),
        ),
    )(x, wbig, bbig, sel)
    return out
